# Initial kernel scaffold; baseline (speedup 1.0000x reference)
import jax
import jax.numpy as jnp
from jax.experimental import pallas as pl


def kernel(queries, keys, k):
    raise NotImplementedError("write your pallas kernel here")


# speedup vs baseline: 19.0133x; 19.0133x over previous
import functools

import jax
import jax.numpy as jnp
from jax import lax
from jax.experimental import pallas as pl
from jax.experimental.pallas import tpu as pltpu
from jax.experimental.pallas import tpu_sc as plsc

Q = 512
D = 256
N = 65536
K = 16
BK = 4096
NB = N // BK


def _topk_body(qref, kref, od_ref, oi_ref, rv_ref, ri_ref):
    j = pl.program_id(0)

    @pl.when(j == 0)
    def _init():
        rv_ref[...] = jnp.full((Q, K), jnp.inf, dtype=jnp.float32)
        ri_ref[...] = jnp.zeros((Q, K), dtype=jnp.int32)

    q = qref[...]
    kb = kref[...]
    q_sq = jnp.sum(q * q, axis=1, keepdims=True)
    k_sq = jnp.sum(kb * kb, axis=1)[None, :]
    mm = lax.dot_general(q, kb, (((1,), (1,)), ((), ())),
                         preferred_element_type=jnp.float32)
    d2 = q_sq + k_sq - 2.0 * mm
    dist = jnp.sqrt(jnp.maximum(d2, 1e-12))

    iota = lax.broadcasted_iota(jnp.int32, (Q, BK), 1)
    base = j * BK
    s = dist
    bvals, bidx = [], []
    for _ in range(K):
        m = jnp.min(s, axis=1, keepdims=True)
        cand = jnp.where(s == m, iota, jnp.int32(N))
        ix = jnp.min(cand, axis=1, keepdims=True)
        s = jnp.where(iota == ix, jnp.float32(jnp.inf), s)
        bvals.append(m)
        bidx.append(ix + base)
    bv = jnp.concatenate(bvals, axis=1)
    bi = jnp.concatenate(bidx, axis=1)

    mv = jnp.concatenate([rv_ref[...], bv], axis=1)
    mi = jnp.concatenate([ri_ref[...], bi], axis=1)
    nv, ni = [], []
    for _ in range(K):
        m = jnp.min(mv, axis=1, keepdims=True)
        ci = jnp.min(jnp.where(mv == m, mi, jnp.int32(2 * N)),
                     axis=1, keepdims=True)
        mask = (mv == m) & (mi == ci)
        mv = jnp.where(mask, jnp.float32(jnp.inf), mv)
        nv.append(m)
        ni.append(ci)
    new_v = jnp.concatenate(nv, axis=1)
    new_i = jnp.concatenate(ni, axis=1)
    rv_ref[...] = new_v
    ri_ref[...] = new_i

    @pl.when(j == NB - 1)
    def _done():
        od_ref[...] = new_v
        oi_ref[...] = new_i


def _topk_call(queries, keys, interpret=False):
    return pl.pallas_call(
        _topk_body,
        grid=(NB,),
        in_specs=[
            pl.BlockSpec((Q, D), lambda j: (0, 0)),
            pl.BlockSpec((BK, D), lambda j: (j, 0)),
        ],
        out_specs=[
            pl.BlockSpec((Q, K), lambda j: (0, 0)),
            pl.BlockSpec((Q, K), lambda j: (0, 0)),
        ],
        out_shape=[
            jax.ShapeDtypeStruct((Q, K), jnp.float32),
            jax.ShapeDtypeStruct((Q, K), jnp.int32),
        ],
        scratch_shapes=[
            pltpu.VMEM((Q, K), jnp.float32),
            pltpu.VMEM((Q, K), jnp.int32),
        ],
        compiler_params=pltpu.CompilerParams(
            dimension_semantics=("arbitrary",),
        ),
        interpret=interpret,
    )(queries, keys)


def _make_sc_gather():
    info = plsc.get_sparse_core_info()
    nw = info.num_cores * info.num_subcores
    b_per_w = Q // nw
    mesh = plsc.VectorSubcoreMesh(core_axis_name="c", subcore_axis_name="s")

    @functools.partial(
        pl.kernel,
        mesh=mesh,
        out_type=jax.ShapeDtypeStruct((Q, D), jnp.float32),
        scratch_types=[
            pltpu.VMEM((b_per_w,), jnp.int32),
            pltpu.VMEM((b_per_w, D), jnp.float32),
            pltpu.SemaphoreType.DMA,
        ],
    )
    def _gather(table_hbm, idx_hbm, out_hbm, idx_v, rows_v, sem):
        wid = lax.axis_index("s") * info.num_cores + lax.axis_index("c")
        base = wid * b_per_w
        pltpu.sync_copy(idx_hbm.at[pl.ds(base, b_per_w)], idx_v)
        pltpu.async_copy(table_hbm.at[idx_v], rows_v, sem).wait()
        pltpu.sync_copy(rows_v, out_hbm.at[pl.ds(base, b_per_w)])

    return _gather


def kernel(queries, keys, k):
    del k
    topk_dists, topk_idx = _topk_call(queries, keys)
    nearest = _make_sc_gather()(keys, topk_idx[:, 0])
    return (topk_dists, topk_idx, nearest)
